# baseline (speedup 1.0000x reference)
import functools

import jax
import jax.numpy as jnp
from jax import lax
from jax.experimental import pallas as pl
from jax.experimental.pallas import tpu as pltpu
from jax.experimental.pallas import tpu_sc as plsc

_NC = 2
_NS = 16
_NW = _NC * _NS
_L = 16


def _sc_partials(n2, e, p):
    gtot = e // _L
    gsmall = gtot // _NW
    rem = gtot % _NW
    gbig = gsmall + (1 if rem else 0)
    epw = (gsmall + 1) * _L
    mesh = plsc.VectorSubcoreMesh(core_axis_name="c", subcore_axis_name="s")

    @functools.partial(
        pl.kernel,
        out_type=jax.ShapeDtypeStruct((_NW, 5, _L), jnp.float32),
        mesh=mesh,
        compiler_params=pltpu.CompilerParams(needs_layout_passes=False),
        scratch_types=[
            pltpu.VMEM((n2,), jnp.float32),
            pltpu.VMEM((epw,), jnp.int32),
            pltpu.VMEM((epw,), jnp.int32),
            pltpu.VMEM((epw,), jnp.float32),
            pltpu.VMEM((p * epw,), jnp.float32),
            pltpu.VMEM((5, _L), jnp.float32),
        ],
    )
    def sc_kernel(vtab_hbm, ei_hbm, w_hbm, par_hbm, out_hbm,
                  vtab_v, src_v, dst_v, w_v, par_v, res_v):
        wid = lax.axis_index("s") * _NC + lax.axis_index("c")
        nsmall = _NW - rem if rem else _NW
        is_small = wid < nsmall
        base = jnp.where(is_small, wid * gsmall,
                         nsmall * gsmall + (wid - nsmall) * gbig) * _L
        ne = gbig * _L if rem else gsmall * _L

        pltpu.sync_copy(vtab_hbm, vtab_v)
        pltpu.sync_copy(ei_hbm.at[pl.ds(base, ne)], src_v.at[pl.ds(0, ne)])
        pltpu.sync_copy(ei_hbm.at[pl.ds(e + base, ne)],
                        dst_v.at[pl.ds(0, ne)])
        pltpu.sync_copy(w_hbm.at[pl.ds(base, ne)], w_v.at[pl.ds(0, ne)])
        pltpu.sync_copy(par_hbm.at[pl.ds(base * p, ne * p)],
                        par_v.at[pl.ds(0, ne * p)])
        if rem:
            @pl.when(is_small)
            def _():
                w_v[pl.ds(gsmall * _L, _L)] = jnp.zeros((_L,), jnp.float32)

        half = jnp.float32(0.5)
        th = jnp.float32(1.5)
        eps = jnp.float32(1e-12)
        pat = lax.iota(jnp.int32, _L) // p

        def body(g, carry):
            off = g * _L
            s2i = src_v[pl.ds(off, _L)] * 2
            d2i = dst_v[pl.ds(off, _L)] * 2
            vrs = plsc.load_gather(vtab_v, [s2i])
            vis = plsc.load_gather(vtab_v, [s2i + 1])
            vrd = plsc.load_gather(vtab_v, [d2i])
            vid = plsc.load_gather(vtab_v, [d2i + 1])
            w = w_v[pl.ds(off, _L)]
            dr = vrs - vrd
            di = vis - vid
            x = dr * dr + di * di + eps
            yi = 0x5F3759DF - lax.shift_right_logical(
                plsc.bitcast(x, jnp.int32), 1)
            y = plsc.bitcast(yi, jnp.float32)
            hx = half * x
            y = y * (th - hx * y * y)
            y = y * (th - hx * y * y)
            y = y * (th - hx * y * y)
            vd = x * y * w
            wa, t1, t2, s1, s2 = carry
            lpg = _L // p
            for j in range(p):
                wj = plsc.load_gather(w_v, [off + j * lpg + pat])
                pv = par_v[pl.ds(off * p + j * _L, _L)]
                pw = pv * wj
                s1 = s1 + pw
                s2 = s2 + pv * pw
            return (wa + w, t1 + vd, t2 + vd * vd, s1, s2)

        zero = jnp.zeros((_L,), jnp.float32)
        init = (zero, zero, zero, zero, zero)
        wa, t1, t2, s1, s2 = lax.fori_loop(0, gbig if rem else gsmall,
                                           body, init)
        res_v[0, :] = wa
        res_v[1, :] = t1
        res_v[2, :] = t2
        res_v[3, :] = s1
        res_v[4, :] = s2
        pltpu.sync_copy(res_v, out_hbm.at[wid])

    return sc_kernel


def _tc_finish(e, p):
    ef = float(e)

    def body(x_ref, o_ref):
        x = x_ref[...]
        s0 = jnp.sum(x[:, 0 * _L:1 * _L])
        t1 = jnp.sum(x[:, 1 * _L:2 * _L])
        t2 = jnp.sum(x[:, 2 * _L:3 * _L])
        s1v = x[:, 3 * _L:4 * _L]
        s2v = x[:, 4 * _L:5 * _L]
        lane = lax.broadcasted_iota(jnp.int32, (_NW, _L), 1) % p
        denom = s0 + jnp.float32(1e-6)
        acc = jnp.float32(0.0)
        zero = jnp.zeros((_NW, _L), jnp.float32)
        for j in range(p):
            s1 = jnp.sum(jnp.where(lane == j, s1v, zero))
            s2 = jnp.sum(jnp.where(lane == j, s2v, zero))
            m = s1 / denom
            acc = acc + (s2 - 2.0 * m * s1 + m * m * s0)
        pc = acc / jnp.float32(p)
        vc = (t2 - t1 * t1 / jnp.float32(ef)) / jnp.float32(ef - 1.0)
        o_ref[0, 0] = pc + vc

    return pl.pallas_call(
        body,
        out_shape=jax.ShapeDtypeStruct((1, 1), jnp.float32),
        out_specs=pl.BlockSpec(memory_space=pltpu.SMEM),
    )


def kernel(node_features, edge_index, edge_probs, edge_params):
    n = node_features.shape[0]
    e = edge_index.shape[1]
    p = edge_params.shape[1]
    assert _L % p == 0, "params per edge must divide the SC lane count"
    if e % _L:
        pad = _L - e % _L
        edge_index = jnp.pad(edge_index, ((0, 0), (0, pad)))
        edge_probs = jnp.pad(edge_probs, (0, pad))
        edge_params = jnp.pad(edge_params, ((0, pad), (0, 0)))
    vtab = node_features[:, :2].reshape(-1)
    partials = _sc_partials(2 * n, edge_index.shape[1], p)(
        vtab, edge_index.reshape(-1), edge_probs, edge_params.reshape(-1))
    out = _tc_finish(e, p)(partials.reshape(_NW, 5 * _L))
    return out[0, 0]


# speedup vs baseline: 4.5638x; 4.5638x over previous
import functools

import jax
import jax.numpy as jnp
from jax import lax
from jax.experimental import pallas as pl
from jax.experimental.pallas import tpu as pltpu
from jax.experimental.pallas import tpu_sc as plsc

_NC = 2
_NS = 16
_NW = _NC * _NS
_L = 16


def _sc_partials(n2, e, p, interleaved):
    gtot = e // _L
    gsmall = gtot // _NW
    rem = gtot % _NW
    gbig = gsmall + (1 if rem else 0)
    epw = gbig * _L
    nrows = 3 + 2 * p
    sb = 128
    nsb = (epw + sb - 1) // sb + 1
    tot_sb = e // sb
    mesh = plsc.VectorSubcoreMesh(core_axis_name="c", subcore_axis_name="s")

    if interleaved:
        idx_scratch = [pltpu.VMEM((2, sb * nsb), jnp.int32)]
        par_scratch = pltpu.VMEM((p, sb * nsb), jnp.float32)
    else:
        idx_scratch = [pltpu.VMEM((epw,), jnp.int32),
                       pltpu.VMEM((epw,), jnp.int32)]
        par_scratch = pltpu.VMEM((p * epw,), jnp.float32)

    @functools.partial(
        pl.kernel,
        out_type=jax.ShapeDtypeStruct((_NW, nrows, _L), jnp.float32),
        mesh=mesh,
        compiler_params=pltpu.CompilerParams(needs_layout_passes=False),
        scratch_types=idx_scratch + [
            pltpu.VMEM((n2,), jnp.float32),
            pltpu.VMEM((epw,), jnp.float32),
            par_scratch,
            pltpu.VMEM((nrows, _L), jnp.float32),
            pltpu.SemaphoreType.DMA,
        ],
    )
    def sc_kernel(vtab_hbm, ei_hbm, w_hbm, par_hbm, out_hbm,
                  *refs):
        if interleaved:
            sd_v, vtab_v, w_v, par_v, res_v, sem = refs
        else:
            src_v, dst_v, vtab_v, w_v, par_v, res_v, sem = refs
        wid = lax.axis_index("s") * _NC + lax.axis_index("c")
        nsmall = _NW - rem if rem else _NW
        is_small = wid < nsmall
        base = jnp.where(is_small, wid * gsmall,
                         nsmall * gsmall + (wid - nsmall) * gbig) * _L

        if interleaved:
            sb0 = jnp.maximum(jnp.minimum(base // sb, tot_sb - nsb), 0)
            rphase = base - sb0 * sb
            var_copies = [
                pltpu.async_copy(ei_hbm.at[:, pl.ds(sb * sb0, sb * nsb)],
                                 sd_v, sem),
                pltpu.async_copy(par_hbm.at[:, pl.ds(sb * sb0, sb * nsb)],
                                 par_v, sem),
            ]
        else:
            var_copies = [
                pltpu.async_copy(ei_hbm.at[pl.ds(base, epw)], src_v, sem),
                pltpu.async_copy(ei_hbm.at[pl.ds(e + base, epw)], dst_v, sem),
            ] + [
                pltpu.async_copy(par_hbm.at[pl.ds(j * e + base, epw)],
                                 par_v.at[pl.ds(j * epw, epw)], sem)
                for j in range(p)
            ]
        copies = var_copies + [
            pltpu.async_copy(vtab_hbm, vtab_v, sem),
            pltpu.async_copy(w_hbm.at[pl.ds(base, epw)], w_v, sem),
        ]
        for c in copies:
            c.wait()
        if rem:
            @pl.when(is_small)
            def _():
                w_v[pl.ds(gsmall * _L, _L)] = jnp.zeros((_L,), jnp.float32)

        half = jnp.float32(0.5)
        th = jnp.float32(1.5)
        eps = jnp.float32(1e-12)

        def body(g, carry):
            off = g * _L
            if interleaved:
                q = rphase + off
                s2i = sd_v[0, pl.ds(q, _L)] * 2
                d2i = sd_v[1, pl.ds(q, _L)] * 2
            else:
                s2i = src_v[pl.ds(off, _L)] * 2
                d2i = dst_v[pl.ds(off, _L)] * 2
            vrs = plsc.load_gather(vtab_v, [s2i])
            vis = plsc.load_gather(vtab_v, [s2i + 1])
            vrd = plsc.load_gather(vtab_v, [d2i])
            vid = plsc.load_gather(vtab_v, [d2i + 1])
            w = w_v[pl.ds(off, _L)]
            dr = vrs - vrd
            di = vis - vid
            x = dr * dr + di * di + eps
            yi = 0x5F3759DF - lax.shift_right_logical(
                plsc.bitcast(x, jnp.int32), 1)
            y = plsc.bitcast(yi, jnp.float32)
            hx = half * x
            y = y * (th - hx * y * y)
            y = y * (th - hx * y * y)
            y = y * (th - hx * y * y)
            vd = x * y * w
            wa, t1, t2, s1, s2 = carry
            ns1 = []
            ns2 = []
            for j in range(p):
                if interleaved:
                    pv = par_v[j, pl.ds(q, _L)]
                else:
                    pv = par_v[pl.ds(j * epw + off, _L)]
                pw = pv * w
                ns1.append(s1[j] + pw)
                ns2.append(s2[j] + pv * pw)
            return (wa + w, t1 + vd, t2 + vd * vd, tuple(ns1), tuple(ns2))

        zero = jnp.zeros((_L,), jnp.float32)
        init = (zero, zero, zero, (zero,) * p, (zero,) * p)
        wa, t1, t2, s1, s2 = lax.fori_loop(0, gbig, body, init, unroll=2)
        res_v[0, :] = wa
        res_v[1, :] = t1
        res_v[2, :] = t2
        for j in range(p):
            res_v[3 + j, :] = s1[j]
            res_v[3 + p + j, :] = s2[j]
        pltpu.sync_copy(res_v, out_hbm.at[wid])

    return sc_kernel


def _tc_finish(e, p):
    ef = float(e)

    def body(x_ref, o_ref):
        s0 = jnp.sum(x_ref[:, 0, :])
        t1 = jnp.sum(x_ref[:, 1, :])
        t2 = jnp.sum(x_ref[:, 2, :])
        denom = s0 + jnp.float32(1e-6)
        acc = jnp.float32(0.0)
        for j in range(p):
            s1 = jnp.sum(x_ref[:, 3 + j, :])
            s2 = jnp.sum(x_ref[:, 3 + p + j, :])
            m = s1 / denom
            acc = acc + (s2 - 2.0 * m * s1 + m * m * s0)
        pc = acc / jnp.float32(p)
        vc = (t2 - t1 * t1 / jnp.float32(ef)) / jnp.float32(ef - 1.0)
        o_ref[0, 0] = pc + vc

    return pl.pallas_call(
        body,
        out_shape=jax.ShapeDtypeStruct((1, 1), jnp.float32),
        out_specs=pl.BlockSpec(memory_space=pltpu.SMEM),
    )


def kernel(node_features, edge_index, edge_probs, edge_params):
    n = node_features.shape[0]
    e = edge_index.shape[1]
    p = edge_params.shape[1]
    if e % _L:
        pad = _L - e % _L
        edge_index = jnp.pad(edge_index, ((0, 0), (0, pad)))
        edge_probs = jnp.pad(edge_probs, (0, pad))
        edge_params = jnp.pad(edge_params, ((0, pad), (0, 0)))
    vtab = node_features[:, :2].reshape(-1)
    ep = edge_index.shape[1]
    interleaved = ep % 128 == 0
    if interleaved:
        ei_lin = edge_index
        par_lin = edge_params.T
    else:
        ei_lin = edge_index.reshape(-1)
        par_lin = edge_params.T.reshape(-1)
    partials = _sc_partials(2 * n, ep, p, interleaved)(
        vtab, ei_lin, edge_probs, par_lin)
    return _tc_finish(e, p)(partials)[0, 0]
